# Initial kernel scaffold; baseline (speedup 1.0000x reference)
import jax
import jax.numpy as jnp
from jax.experimental import pallas as pl


def kernel(x, edge_index, batch, W_ip, b_ip, W_ih_enc, W_hh_enc, b_ih_enc, b_hh_enc, W_dyn, b_dyn, W_g1, b_g1, W_g2, b_g2, W_nb, b_nb, W_ih0, W_hh0, b_ih0, b_hh0, W_ih1, W_hh1, b_ih1, b_hh1, W_op, b_op):
    raise NotImplementedError("write your pallas kernel here")


# speedup vs baseline: 3.1587x; 3.1587x over previous
import functools

import jax
import jax.numpy as jnp
from jax.experimental import pallas as pl
from jax.experimental.pallas import tpu as pltpu

N = 50000
E = 800000
NG = 2500
T = 10
IE = 32
H = 64
DEC = 128
OUT_LEN = 25

ENC_BLK = 2000


def _leaky(v):
    return jnp.where(v > 0, v, 0.1 * v)


def _encoder_body(x_ref, Wbig_ref, bbig_ref, WihT_ref, WhhT_ref, bih_ref,
                  bhh_ref, Wdyn_ref, bdyn_ref, hist_ref):
    xb = x_ref[...]
    B = xb.shape[0]
    emb = _leaky(jnp.dot(xb, Wbig_ref[...],
                         preferred_element_type=jnp.float32) + bbig_ref[...])
    WihT = WihT_ref[...]
    WhhT = WhhT_ref[...]
    bih = bih_ref[...]
    bhh = bhh_ref[...]
    h = jnp.zeros((B, H), jnp.float32)
    for t in range(T):
        xt = emb[:, t * IE:(t + 1) * IE]
        gi = jnp.dot(xt, WihT, preferred_element_type=jnp.float32) + bih
        gh = jnp.dot(h, WhhT, preferred_element_type=jnp.float32) + bhh
        r = jax.nn.sigmoid(gi[:, :H] + gh[:, :H])
        z = jax.nn.sigmoid(gi[:, H:2 * H] + gh[:, H:2 * H])
        nn_ = jnp.tanh(gi[:, 2 * H:] + r * gh[:, 2 * H:])
        h = (1 - z) * nn_ + z * h
    hist_ref[...] = _leaky(
        jnp.dot(_leaky(h), Wdyn_ref[...],
                preferred_element_type=jnp.float32) + bdyn_ref[...])


def _run_encoder(x2d, W_ip, b_ip, W_ih_enc, W_hh_enc, b_ih_enc, b_hh_enc,
                 W_dyn, b_dyn):
    Wbig = jnp.zeros((T * 2, T * IE), jnp.float32)
    for t in range(T):
        Wbig = jax.lax.dynamic_update_slice(Wbig, W_ip, (2 * t, IE * t))
    bbig = jnp.tile(b_ip, T)
    grid = (N // ENC_BLK,)
    full = lambda shape: pl.BlockSpec(shape, lambda i: (0,) * len(shape))
    return pl.pallas_call(
        _encoder_body,
        grid=grid,
        in_specs=[
            pl.BlockSpec((ENC_BLK, T * 2), lambda i: (i, 0)),
            full((T * 2, T * IE)),
            full((T * IE,)),
            full((IE, 3 * H)),
            full((H, 3 * H)),
            full((3 * H,)),
            full((3 * H,)),
            full((H, H)),
            full((H,)),
        ],
        out_specs=pl.BlockSpec((ENC_BLK, H), lambda i: (i, 0)),
        out_shape=jax.ShapeDtypeStruct((N, H), jnp.float32),
    )(x2d, Wbig, bbig, W_ih_enc.T, W_hh_enc.T, b_ih_enc, b_hh_enc, W_dyn,
      b_dyn)


def _decoder_body(g2t_ref, histt_ref, Wnb_ref, bnb_ref, Wih0T_ref, Whh0T_ref,
                  b0_ref, Wih1T_ref, Whh1T_ref, b1_ref, Wop_ref, bop_ref,
                  out_ref, enc_gi, h0s, c0s, h1s, c1s):
    t = pl.program_id(0)

    @pl.when(t == 0)
    def _init():
        tgt = jnp.concatenate([g2t_ref[...], histt_ref[...]], axis=1)
        gat = _leaky(jnp.dot(tgt, Wnb_ref[...],
                             preferred_element_type=jnp.float32) + bnb_ref[...])
        enc = jnp.concatenate([histt_ref[...], gat], axis=1)
        enc_gi[...] = jnp.dot(enc, Wih0T_ref[...],
                              preferred_element_type=jnp.float32) + b0_ref[...]
        h0s[...] = jnp.zeros_like(h0s)
        c0s[...] = jnp.zeros_like(c0s)
        h1s[...] = jnp.zeros_like(h1s)
        c1s[...] = jnp.zeros_like(c1s)

    g = enc_gi[...] + jnp.dot(h0s[...], Whh0T_ref[...],
                              preferred_element_type=jnp.float32)
    i = jax.nn.sigmoid(g[:, :DEC])
    f = jax.nn.sigmoid(g[:, DEC:2 * DEC])
    gg = jnp.tanh(g[:, 2 * DEC:3 * DEC])
    o = jax.nn.sigmoid(g[:, 3 * DEC:])
    c0 = f * c0s[...] + i * gg
    h0 = o * jnp.tanh(c0)
    c0s[...] = c0
    h0s[...] = h0

    g = (jnp.dot(h0, Wih1T_ref[...], preferred_element_type=jnp.float32) +
         b1_ref[...] +
         jnp.dot(h1s[...], Whh1T_ref[...], preferred_element_type=jnp.float32))
    i = jax.nn.sigmoid(g[:, :DEC])
    f = jax.nn.sigmoid(g[:, DEC:2 * DEC])
    gg = jnp.tanh(g[:, 2 * DEC:3 * DEC])
    o = jax.nn.sigmoid(g[:, 3 * DEC:])
    c1 = f * c1s[...] + i * gg
    h1 = o * jnp.tanh(c1)
    c1s[...] = c1
    h1s[...] = h1

    out_ref[...] = (jnp.dot(h1, Wop_ref[...],
                            preferred_element_type=jnp.float32) +
                    bop_ref[...])[None]


def _run_decoder(g2_t, hist_t, W_nb, b_nb, W_ih0, W_hh0, b_ih0, b_hh0, W_ih1,
                 W_hh1, b_ih1, b_hh1, W_op, b_op):
    full = lambda shape: pl.BlockSpec(shape, lambda i: (0,) * len(shape))
    out = pl.pallas_call(
        _decoder_body,
        grid=(OUT_LEN,),
        in_specs=[
            full((NG, H)),
            full((NG, H)),
            full((2 * H, H)),
            full((H,)),
            full((2 * H, 4 * DEC)),
            full((DEC, 4 * DEC)),
            full((4 * DEC,)),
            full((DEC, 4 * DEC)),
            full((DEC, 4 * DEC)),
            full((4 * DEC,)),
            full((DEC, 2)),
            full((2,)),
        ],
        out_specs=pl.BlockSpec((1, NG, 2), lambda t: (t, 0, 0)),
        out_shape=jax.ShapeDtypeStruct((OUT_LEN, NG, 2), jnp.float32),
        scratch_shapes=[
            pltpu.VMEM((NG, 4 * DEC), jnp.float32),
            pltpu.VMEM((NG, DEC), jnp.float32),
            pltpu.VMEM((NG, DEC), jnp.float32),
            pltpu.VMEM((NG, DEC), jnp.float32),
            pltpu.VMEM((NG, DEC), jnp.float32),
        ],
    )(g2_t, hist_t, W_nb, b_nb, W_ih0.T, W_hh0.T, b_ih0 + b_hh0, W_ih1.T,
      W_hh1.T, b_ih1 + b_hh1, W_op, b_op)
    return jnp.transpose(out, (1, 0, 2))


def kernel(x, edge_index, batch, W_ip, b_ip, W_ih_enc, W_hh_enc, b_ih_enc,
           b_hh_enc, W_dyn, b_dyn, W_g1, b_g1, W_g2, b_g2, W_nb, b_nb, W_ih0,
           W_hh0, b_ih0, b_hh0, W_ih1, W_hh1, b_ih1, b_hh1, W_op, b_op):
    src, dst = edge_index[0], edge_index[1]
    target_index = jnp.searchsorted(batch, jnp.arange(NG, dtype=batch.dtype))

    hist = _run_encoder(x.reshape(N, T * 2), W_ip, b_ip, W_ih_enc, W_hh_enc,
                        b_ih_enc, b_hh_enc, W_dyn, b_dyn)

    deg = jax.ops.segment_sum(jnp.ones(E, jnp.float32), dst, N) + 1.0
    dinv = deg ** -0.5

    def conv_edge_sum(xw):
        rows = xw * dinv[:, None]
        return jax.ops.segment_sum(rows[src], dst, N)

    xw1 = jnp.dot(hist, W_g1)
    g1 = dinv[:, None] * (conv_edge_sum(xw1) + dinv[:, None] * xw1) + b_g1
    xw2 = jnp.dot(g1, W_g2[:H]) + jnp.dot(hist, W_g2[H:])
    g2 = dinv[:, None] * (conv_edge_sum(xw2) + dinv[:, None] * xw2) + b_g2

    g2_t = g2[target_index]
    hist_t = hist[target_index]

    return _run_decoder(g2_t, hist_t, W_nb, b_nb, W_ih0, W_hh0, b_ih0, b_hh0,
                        W_ih1, W_hh1, b_ih1, b_hh1, W_op, b_op)


# speedup vs baseline: 11.8091x; 3.7386x over previous
import functools

import jax
import jax.numpy as jnp
from jax import lax
from jax.experimental import pallas as pl
from jax.experimental.pallas import tpu as pltpu
from jax.experimental.pallas import tpu_sc as plsc

N = 50000
E = 800000
NG = 2500
T = 10
IE = 32
H = 64
DEC = 128
OUT_LEN = 25

ENC_BLK = 2000

NC = 2
NS = 16
L = 16

NPAD = 51200
NTAB = N + 16
PAD_E = 819200
ER = PAD_E // 128
DEG_ROWS_PER_TILE = ER // (NC * NS)
CONV_ROWS_PER_TILE = ER // NS
ZROWS = NPAD // NS


def _leaky(v):
    return jnp.where(v > 0, v, 0.1 * v)


def _sc_mesh():
    return plsc.VectorSubcoreMesh(core_axis_name="c", subcore_axis_name="s",
                                  num_cores=NC, num_subcores=NS)


NL = 50176
DEG_CHUNK = 1024
DEG_CHUNKS_PER_TILE = PAD_E // (NC * NS * DEG_CHUNK)


@functools.partial(
    pl.kernel,
    out_type=jax.ShapeDtypeStruct((NC * NS, NL), jnp.float32),
    mesh=_sc_mesh(),
    scratch_types=[
        pltpu.VMEM((DEG_CHUNK,), jnp.int32),
        pltpu.VMEM((NL,), jnp.float32),
    ],
    compiler_params=pltpu.CompilerParams(needs_layout_passes=False),
)
def _deg_kernel(dst_hbm, out_hbm, idx_d, dloc):
    c = lax.axis_index("c")
    s = lax.axis_index("s")
    w = c * NS + s

    def fill_z(i, _):
        dloc[pl.ds(i * L, L)] = jnp.zeros((L,), jnp.float32)
        return 0

    lax.fori_loop(0, NL // L, fill_z, 0)

    ones16 = jnp.ones((L,), jnp.float32)
    ebase = w * DEG_CHUNKS_PER_TILE * DEG_CHUNK

    def body(ch, _):
        pltpu.sync_copy(dst_hbm.at[pl.ds(ebase + ch * DEG_CHUNK, DEG_CHUNK)],
                        idx_d)

        def inner(g, _):
            idx16 = idx_d[pl.ds(g * L, L)]
            plsc.addupdate_scatter(dloc, [idx16], ones16)
            return 0

        lax.fori_loop(0, DEG_CHUNK // L, inner, 0)
        return 0

    lax.fori_loop(0, DEG_CHUNKS_PER_TILE, body, 0)
    pltpu.sync_copy(dloc, out_hbm.at[w])


FQ = 16
SUP = 16
CONV_ITERS = CONV_ROWS_PER_TILE // SUP


@functools.partial(
    pl.kernel,
    out_type=jax.ShapeDtypeStruct((2 * NPAD, FQ), jnp.float32),
    mesh=_sc_mesh(),
    scratch_types=[
        pltpu.VMEM((SUP, 128), jnp.int32),
        pltpu.VMEM((SUP, 128), jnp.int32),
        pltpu.VMEM((SUP, 128, FQ), jnp.float32),
        pltpu.VMEM((800, FQ), jnp.float32),
        pltpu.VMEM_SHARED((NPAD, FQ), jnp.float32),
        pltpu.SemaphoreType.DMA,
        pltpu.SemaphoreType.DMA,
    ],
    compiler_params=pltpu.CompilerParams(needs_layout_passes=False,
                                         use_tc_tiling_on_sc=False),
)
def _conv_kernel(src_hbm, dst_hbm, qa_hbm, qb_hbm, out_hbm, idx_s, idx_d,
                 rows, zbuf, shared, gsem, ssem):
    c = lax.axis_index("c")
    s = lax.axis_index("s")

    def fill_z(i, _):
        zbuf[i] = jnp.zeros((FQ,), jnp.float32)
        return 0

    lax.fori_loop(0, 800, fill_z, 0)
    for k in range(ZROWS // 800):
        pltpu.sync_copy(zbuf, shared.at[pl.ds(s * ZROWS + k * 800, 800)])
    plsc.subcore_barrier()

    rbase = s * CONV_ROWS_PER_TILE

    def body(ch, _):
        r0 = rbase + ch * SUP
        pltpu.sync_copy(src_hbm.at[pl.ds(r0, SUP)], idx_s)
        pltpu.sync_copy(dst_hbm.at[pl.ds(r0, SUP)], idx_d)

        @pl.when(c == 0)
        def _():
            hs = [pltpu.async_copy(qa_hbm.at[idx_s.at[j]], rows.at[j], gsem)
                  for j in range(SUP)]
            for h in hs:
                h.wait()

        @pl.when(c == 1)
        def _():
            hs = [pltpu.async_copy(qb_hbm.at[idx_s.at[j]], rows.at[j], gsem)
                  for j in range(SUP)]
            for h in hs:
                h.wait()

        hs = [pltpu.async_copy(rows.at[j], shared.at[idx_d.at[j]], ssem,
                               add=True) for j in range(SUP)]
        for h in hs:
            h.wait()
        return 0

    lax.fori_loop(0, CONV_ITERS, body, 0)
    plsc.subcore_barrier()
    pltpu.sync_copy(shared.at[pl.ds(s * ZROWS, ZROWS)],
                    out_hbm.at[pl.ds(c * NPAD + s * ZROWS, ZROWS)])


def _encoder_body(x_ref, dinv_ref, Wbig_ref, bbig_ref, WihT_ref, WhhT_ref,
                  bih_ref, bhh_ref, Wdyn_ref, bdyn_ref, Wg1_ref, hist_ref,
                  q0_ref, q1_ref, q2_ref, q3_ref):
    xb = x_ref[...]
    B = xb.shape[0]
    emb = _leaky(jnp.dot(xb, Wbig_ref[...],
                         preferred_element_type=jnp.float32) + bbig_ref[...])
    WihT = WihT_ref[...]
    WhhT = WhhT_ref[...]
    bih = bih_ref[...]
    bhh = bhh_ref[...]
    h = jnp.zeros((B, H), jnp.float32)
    for t in range(T):
        xt = emb[:, t * IE:(t + 1) * IE]
        gi = jnp.dot(xt, WihT, preferred_element_type=jnp.float32) + bih
        gh = jnp.dot(h, WhhT, preferred_element_type=jnp.float32) + bhh
        r = jax.nn.sigmoid(gi[:, :H] + gh[:, :H])
        z = jax.nn.sigmoid(gi[:, H:2 * H] + gh[:, H:2 * H])
        nn_ = jnp.tanh(gi[:, 2 * H:] + r * gh[:, 2 * H:])
        h = (1 - z) * nn_ + z * h
    hist = _leaky(jnp.dot(_leaky(h), Wdyn_ref[...],
                          preferred_element_type=jnp.float32) + bdyn_ref[...])
    hist_ref[...] = hist
    xw1s = jnp.dot(hist, Wg1_ref[...],
                   preferred_element_type=jnp.float32) * dinv_ref[...]
    q0_ref[...] = xw1s[:, 0 * FQ:1 * FQ]
    q1_ref[...] = xw1s[:, 1 * FQ:2 * FQ]
    q2_ref[...] = xw1s[:, 2 * FQ:3 * FQ]
    q3_ref[...] = xw1s[:, 3 * FQ:4 * FQ]


def _run_encoder(x2d, dinv, W_ip, b_ip, W_ih_enc, W_hh_enc, b_ih_enc,
                 b_hh_enc, W_dyn, b_dyn, W_g1):
    Wbig = jnp.zeros((T * 2, T * IE), jnp.float32)
    for t in range(T):
        Wbig = lax.dynamic_update_slice(Wbig, W_ip, (2 * t, IE * t))
    bbig = jnp.tile(b_ip, T)
    full = lambda shape: pl.BlockSpec(shape, lambda i: (0,) * len(shape))
    return pl.pallas_call(
        _encoder_body,
        grid=(N // ENC_BLK,),
        in_specs=[
            pl.BlockSpec((ENC_BLK, T * 2), lambda i: (i, 0)),
            pl.BlockSpec((ENC_BLK, 1), lambda i: (i, 0)),
            full((T * 2, T * IE)),
            full((T * IE,)),
            full((IE, 3 * H)),
            full((H, 3 * H)),
            full((3 * H,)),
            full((3 * H,)),
            full((H, H)),
            full((H,)),
            full((H, H)),
        ],
        out_specs=[pl.BlockSpec((ENC_BLK, H), lambda i: (i, 0))] +
                  [pl.BlockSpec((ENC_BLK, FQ), lambda i: (i, 0))] * 4,
        out_shape=[jax.ShapeDtypeStruct((N, H), jnp.float32)] +
                  [jax.ShapeDtypeStruct((N, FQ), jnp.float32)] * 4,
    )(x2d, dinv, Wbig, bbig, W_ih_enc.T, W_hh_enc.T, b_ih_enc, b_hh_enc,
      W_dyn, b_dyn, W_g1)


def _mid_body(e0_ref, e1_ref, e2_ref, e3_ref, x0_ref, x1_ref, x2_ref,
              x3_ref, hist_ref, dinv_ref, Wg2a_ref, Wg2b_ref, bg1_ref,
              o0_ref, o1_ref, o2_ref, o3_ref):
    dinv = dinv_ref[...]
    es1 = jnp.concatenate([e0_ref[...], e1_ref[...], e2_ref[...],
                           e3_ref[...]], axis=1)
    xw1s = jnp.concatenate([x0_ref[...], x1_ref[...], x2_ref[...],
                            x3_ref[...]], axis=1)
    g1 = dinv * (es1 + xw1s) + bg1_ref[...]
    xw2 = (jnp.dot(g1, Wg2a_ref[...], preferred_element_type=jnp.float32) +
           jnp.dot(hist_ref[...], Wg2b_ref[...],
                   preferred_element_type=jnp.float32))
    xw2s = xw2 * dinv
    o0_ref[...] = xw2s[:, 0 * FQ:1 * FQ]
    o1_ref[...] = xw2s[:, 1 * FQ:2 * FQ]
    o2_ref[...] = xw2s[:, 2 * FQ:3 * FQ]
    o3_ref[...] = xw2s[:, 3 * FQ:4 * FQ]


def _run_mid(es1q, xw1q, hist, dinv, W_g2, b_g1):
    full = lambda shape: pl.BlockSpec(shape, lambda i: (0,) * len(shape))
    blk = lambda w: pl.BlockSpec((ENC_BLK, w), lambda i: (i, 0))
    return pl.pallas_call(
        _mid_body,
        grid=(N // ENC_BLK,),
        in_specs=[blk(FQ)] * 8 + [
            blk(H),
            pl.BlockSpec((ENC_BLK, 1), lambda i: (i, 0)),
            full((H, H)),
            full((H, H)),
            full((H,)),
        ],
        out_specs=[blk(FQ)] * 4,
        out_shape=[jax.ShapeDtypeStruct((N, FQ), jnp.float32)] * 4,
    )(*es1q, *xw1q, hist, dinv, W_g2[:H], W_g2[H:], b_g1)


def _decoder_body(e0_ref, e1_ref, e2_ref, e3_ref, x0_ref, x1_ref, x2_ref,
                  x3_ref, dinv_ref, histt_ref,
                  bg2_ref, Wnb_ref, bnb_ref, Wih0T_ref, Whh0T_ref, b0_ref,
                  Wih1T_ref, Whh1T_ref, b1_ref, Wop_ref, bop_ref, out_ref,
                  enc_gi, h0s, c0s, h1s, c1s):
    t = pl.program_id(0)

    @pl.when(t == 0)
    def _init():
        es2 = jnp.concatenate([e0_ref[...], e1_ref[...], e2_ref[...],
                               e3_ref[...]], axis=1)
        xw2s = jnp.concatenate([x0_ref[...], x1_ref[...], x2_ref[...],
                                x3_ref[...]], axis=1)
        g2 = dinv_ref[...] * (es2 + xw2s) + bg2_ref[...]
        tgt = jnp.concatenate([g2, histt_ref[...]], axis=1)
        gat = _leaky(jnp.dot(tgt, Wnb_ref[...],
                             preferred_element_type=jnp.float32) + bnb_ref[...])
        enc = jnp.concatenate([histt_ref[...], gat], axis=1)
        enc_gi[...] = jnp.dot(enc, Wih0T_ref[...],
                              preferred_element_type=jnp.float32) + b0_ref[...]
        h0s[...] = jnp.zeros_like(h0s)
        c0s[...] = jnp.zeros_like(c0s)
        h1s[...] = jnp.zeros_like(h1s)
        c1s[...] = jnp.zeros_like(c1s)

    g = enc_gi[...] + jnp.dot(h0s[...], Whh0T_ref[...],
                              preferred_element_type=jnp.float32)
    i = jax.nn.sigmoid(g[:, :DEC])
    f = jax.nn.sigmoid(g[:, DEC:2 * DEC])
    gg = jnp.tanh(g[:, 2 * DEC:3 * DEC])
    o = jax.nn.sigmoid(g[:, 3 * DEC:])
    c0 = f * c0s[...] + i * gg
    h0 = o * jnp.tanh(c0)
    c0s[...] = c0
    h0s[...] = h0

    g = (jnp.dot(h0, Wih1T_ref[...], preferred_element_type=jnp.float32) +
         b1_ref[...] +
         jnp.dot(h1s[...], Whh1T_ref[...], preferred_element_type=jnp.float32))
    i = jax.nn.sigmoid(g[:, :DEC])
    f = jax.nn.sigmoid(g[:, DEC:2 * DEC])
    gg = jnp.tanh(g[:, 2 * DEC:3 * DEC])
    o = jax.nn.sigmoid(g[:, 3 * DEC:])
    c1 = f * c1s[...] + i * gg
    h1 = o * jnp.tanh(c1)
    c1s[...] = c1
    h1s[...] = h1

    out_ref[...] = (jnp.dot(h1, Wop_ref[...],
                            preferred_element_type=jnp.float32) +
                    bop_ref[...])[None]


def _run_decoder(es2q_t, xw2q_t, dinv_t, hist_t, b_g2, W_nb,
                 b_nb, W_ih0, W_hh0, b_ih0, b_hh0, W_ih1, W_hh1, b_ih1, b_hh1,
                 W_op, b_op):
    full = lambda shape: pl.BlockSpec(shape, lambda i: (0,) * len(shape))
    out = pl.pallas_call(
        _decoder_body,
        grid=(OUT_LEN,),
        in_specs=[full((NG, FQ))] * 8 + [
            full((NG, 1)),
            full((NG, H)),
            full((H,)),
            full((2 * H, H)),
            full((H,)),
            full((2 * H, 4 * DEC)),
            full((DEC, 4 * DEC)),
            full((4 * DEC,)),
            full((DEC, 4 * DEC)),
            full((DEC, 4 * DEC)),
            full((4 * DEC,)),
            full((DEC, 2)),
            full((2,)),
        ],
        out_specs=pl.BlockSpec((1, NG, 2), lambda t: (t, 0, 0)),
        out_shape=jax.ShapeDtypeStruct((OUT_LEN, NG, 2), jnp.float32),
        scratch_shapes=[
            pltpu.VMEM((NG, 4 * DEC), jnp.float32),
            pltpu.VMEM((NG, DEC), jnp.float32),
            pltpu.VMEM((NG, DEC), jnp.float32),
            pltpu.VMEM((NG, DEC), jnp.float32),
            pltpu.VMEM((NG, DEC), jnp.float32),
        ],
    )(*es2q_t, *xw2q_t, dinv_t, hist_t, b_g2, W_nb, b_nb,
      W_ih0.T, W_hh0.T, b_ih0 + b_hh0, W_ih1.T, W_hh1.T, b_ih1 + b_hh1, W_op,
      b_op)
    return jnp.transpose(out, (1, 0, 2))


def _pad_table(half):
    return jnp.pad(half, ((0, NTAB - N), (0, 0)))


def kernel(x, edge_index, batch, W_ip, b_ip, W_ih_enc, W_hh_enc, b_ih_enc,
           b_hh_enc, W_dyn, b_dyn, W_g1, b_g1, W_g2, b_g2, W_nb, b_nb, W_ih0,
           W_hh0, b_ih0, b_hh0, W_ih1, W_hh1, b_ih1, b_hh1, W_op, b_op):
    epad = jnp.full((PAD_E - E,), N, jnp.int32)
    src2d = jnp.concatenate([edge_index[0].astype(jnp.int32),
                             epad]).reshape(ER, 128)
    dst1d = jnp.concatenate([edge_index[1].astype(jnp.int32), epad])
    dst2d = dst1d.reshape(ER, 128)
    target_index = jnp.searchsorted(batch, jnp.arange(NG, dtype=batch.dtype))

    degp = _deg_kernel(dst1d)
    deg = jnp.sum(degp[:, :N], axis=0) + 1.0
    dinv = (deg ** -0.5)[:, None]

    hist, *xw1q = _run_encoder(x.reshape(N, T * 2), dinv, W_ip, b_ip,
                               W_ih_enc, W_hh_enc, b_ih_enc, b_hh_enc,
                               W_dyn, b_dyn, W_g1)

    def conv(q):
        e02 = _conv_kernel(src2d, dst2d, _pad_table(q[0]), _pad_table(q[2]))
        e13 = _conv_kernel(src2d, dst2d, _pad_table(q[1]), _pad_table(q[3]))
        return [e02[:N], e13[:N], e02[NPAD:NPAD + N], e13[NPAD:NPAD + N]]

    es1q = conv(xw1q)
    xw2q = _run_mid(es1q, xw1q, hist, dinv, W_g2, b_g1)
    es2q = conv(xw2q)

    tk = lambda a: jnp.take(a, target_index, axis=0)
    return _run_decoder([tk(a) for a in es2q], [tk(a) for a in xw2q],
                        tk(dinv), tk(hist), b_g2, W_nb, b_nb, W_ih0,
                        W_hh0, b_ih0, b_hh0, W_ih1, W_hh1, b_ih1, b_hh1, W_op,
                        b_op)
